# baseline (speedup 1.0000x reference)
import functools

import jax
import jax.numpy as jnp
from jax import lax
from jax.experimental import pallas as pl
from jax.experimental.pallas import tpu as pltpu
from jax.experimental.pallas import tpu_sc as plsc

B = 16384
P = 10
D = 32
R = 14 * P
K = P * D
NC, NS = 2, 16
NW = NC * NS
BSL = B // NW
L = 16
BG = BSL // L


def _sc_gather(tbl_t, x_t):
    mesh = plsc.VectorSubcoreMesh(core_axis_name="c", subcore_axis_name="s")

    @functools.partial(
        pl.kernel,
        mesh=mesh,
        out_type=jax.ShapeDtypeStruct((K, B), jnp.float32),
        scratch_types=[
            pltpu.VMEM((D * R,), jnp.float32),
            pltpu.VMEM((BSL,), jnp.int32),
            pltpu.VMEM((BSL,), jnp.int32),
            pltpu.VMEM((D, BSL), jnp.float32),
            pltpu.VMEM((D, BSL), jnp.float32),
            pltpu.SemaphoreType.DMA,
            pltpu.SemaphoreType.DMA,
            pltpu.SemaphoreType.DMA,
            pltpu.SemaphoreType.DMA,
        ],
        compiler_params=pltpu.CompilerParams(needs_layout_passes=False),
    )
    def k(tbl_hbm, x_hbm, out_hbm, tbl_v, xb0, xb1, rb0, rb1,
          sx0, sx1, so0, so1):
        wid = lax.axis_index("s") * NC + lax.axis_index("c")
        b0w = wid * BSL
        xbufs, rbufs = (xb0, xb1), (rb0, rb1)
        sxs, sos = (sx0, sx1), (so0, so1)

        pltpu.sync_copy(tbl_hbm, tbl_v)

        def load_x(p):
            return pltpu.async_copy(
                x_hbm.at[p, pl.ds(b0w, BSL)], xbufs[p % 2], sxs[p % 2])

        x_pend = load_x(0)
        out_pend = [None, None]
        for p in range(P):
            bp = p % 2
            nxt = load_x(p + 1) if p + 1 < P else None
            x_pend.wait()
            x_pend = nxt
            if out_pend[bp] is not None:
                out_pend[bp].wait()
            xbuf, rows = xbufs[bp], rbufs[bp]

            def cw_of(g):
                return xbuf[pl.ds(g * L, L)] * 10 + p

            def ld_st(cw, vals, gst):
                new = []
                for j in range(D):
                    new.append(plsc.load_gather(tbl_v, [cw + j * R]))
                    if vals is not None:
                        rows[j, pl.ds(gst * L, L)] = vals[j]
                return new

            def store_grp(vals, gst):
                for j in range(D):
                    rows[j, pl.ds(gst * L, L)] = vals[j]

            def group(g, carry):
                cwg, vals = carry
                cw_next = cw_of(g + 1)
                new = ld_st(cwg, vals, g - 1)
                return (cw_next, tuple(new))

            vals0 = ld_st(cw_of(0), None, 0)
            cw_last, vals_last = lax.fori_loop(
                1, BG - 1, group, (cw_of(1), tuple(vals0)))
            tail = ld_st(cw_last, vals_last, BG - 2)
            store_grp(tail, BG - 1)
            out_pend[bp] = pltpu.async_copy(
                rows, out_hbm.at[pl.ds(p * D, D), pl.ds(b0w, BSL)], sos[bp])
        for h in out_pend:
            if h is not None:
                h.wait()

    return k(tbl_t, x_t)


def kernel(x, embed_weight, pe):
    tbl3 = embed_weight[:, None, :] + pe[None, :, :]
    tbl_t = tbl3.transpose(2, 0, 1).reshape(D * R)
    x_t = x.T.astype(jnp.int32)
    out2 = _sc_gather(tbl_t, x_t)
    return out2.reshape(P, D, B).transpose(2, 0, 1)


# speedup vs baseline: 1.0590x; 1.0590x over previous
import functools

import jax
import jax.numpy as jnp
from jax import lax
from jax.experimental import pallas as pl
from jax.experimental.pallas import tpu as pltpu
from jax.experimental.pallas import tpu_sc as plsc

B = 16384
P = 10
D = 32
R = 14 * P
K = P * D
NC, NS = 2, 16
NW = NC * NS
BSL = B // NW
L = 16
BG = BSL // L


def _sc_gather(tbl_t, x_t):
    mesh = plsc.VectorSubcoreMesh(core_axis_name="c", subcore_axis_name="s")

    @functools.partial(
        pl.kernel,
        mesh=mesh,
        out_type=jax.ShapeDtypeStruct((K, B), jnp.float32),
        scratch_types=[
            pltpu.VMEM((D * R,), jnp.float32),
            pltpu.VMEM((BSL,), jnp.int32),
            pltpu.VMEM((BSL,), jnp.int32),
            pltpu.VMEM((D, BSL), jnp.float32),
            pltpu.VMEM((D, BSL), jnp.float32),
            pltpu.SemaphoreType.DMA,
            pltpu.SemaphoreType.DMA,
            pltpu.SemaphoreType.DMA,
            pltpu.SemaphoreType.DMA,
        ],
        compiler_params=pltpu.CompilerParams(needs_layout_passes=False),
    )
    def k(tbl_hbm, x_hbm, out_hbm, tbl_v, xb0, xb1, rb0, rb1,
          sx0, sx1, so0, so1):
        wid = lax.axis_index("s") * NC + lax.axis_index("c")
        b0w = wid * BSL
        xbufs, rbufs = (xb0, xb1), (rb0, rb1)
        sxs, sos = (sx0, sx1), (so0, so1)

        pltpu.sync_copy(tbl_hbm, tbl_v)

        def load_x(p):
            return pltpu.async_copy(
                x_hbm.at[p, pl.ds(b0w, BSL)], xbufs[p % 2], sxs[p % 2])

        x_pend = load_x(0)
        out_pend = [None, None]
        for p in range(P):
            bp = p % 2
            nxt = load_x(p + 1) if p + 1 < P else None
            x_pend.wait()
            x_pend = nxt
            if out_pend[bp] is not None:
                out_pend[bp].wait()
            xbuf, rows = xbufs[bp], rbufs[bp]

            def load_grp(g):
                base16 = xbuf[pl.ds(g * L, L)] * 10 + p
                return [plsc.load_gather(tbl_v, [base16 + j * R])
                        for j in range(D)]

            def group(g, vals):
                base16 = xbuf[pl.ds(g * L, L)] * 10 + p
                new = []
                for j in range(D):
                    new.append(plsc.load_gather(tbl_v, [base16 + j * R]))
                    rows[j, pl.ds((g - 1) * L, L)] = vals[j]
                return tuple(new)

            vals_last = lax.fori_loop(1, BG, group, tuple(load_grp(0)))
            for j in range(D):
                rows[j, pl.ds((BG - 1) * L, L)] = vals_last[j]
            out_pend[bp] = pltpu.async_copy(
                rows, out_hbm.at[pl.ds(p * D, D), pl.ds(b0w, BSL)], sos[bp])
        for h in out_pend:
            if h is not None:
                h.wait()

    return k(tbl_t, x_t)


def kernel(x, embed_weight, pe):
    tbl3 = embed_weight[:, None, :] + pe[None, :, :]
    tbl_t = tbl3.transpose(2, 0, 1).reshape(D * R)
    x_t = x.T.astype(jnp.int32)
    out2 = _sc_gather(tbl_t, x_t)
    return out2.reshape(P, D, B).transpose(2, 0, 1)
